# baseline (speedup 1.0000x reference)
import functools

import jax
import jax.numpy as jnp
from jax import lax
from jax.experimental import pallas as pl
from jax.experimental.pallas import tpu as pltpu
from jax.experimental.pallas import tpu_sc as plsc

S = 4096
LAT = 1024
SPLIT = 3072
N = S * LAT
SPLIT_E = SPLIT * LAT
NW = 32
P = N // NW
C = 65536
NCH = P // C


def _sc_body(hist_ref, hval_ref, scal_ref, out_ref,
             buf, scal_v, hv, acc, row):
    cid = lax.axis_index("c")
    sid = lax.axis_index("s")
    wid = cid * 16 + sid

    pltpu.sync_copy(scal_ref, scal_v)
    idx = scal_v[pl.ds(0, 16)][0]
    ctr = scal_v[pl.ds(1 + 2 * idx, 16)][0]
    ovf = ctr == SPLIT
    base = idx * N

    for c in range(NCH):
        d0 = wid * P + c * C
        is_first = d0 == 0
        is_mean = jnp.logical_and(d0 == SPLIT_E, ovf)
        special = jnp.logical_or(is_first, is_mean)
        shifted = jnp.logical_or(d0 < SPLIT_E, ovf)

        @pl.when(special)
        def _():
            pltpu.sync_copy(hist_ref.at[pl.ds(base + d0, C - LAT)],
                            buf.at[pl.ds(0, C - LAT)])
            pltpu.sync_copy(buf.at[pl.ds(0, C - LAT)],
                            out_ref.at[pl.ds(d0 + LAT, C - LAT)])

        @pl.when(jnp.logical_not(special))
        def _():
            src0 = base + d0 - jnp.where(shifted, LAT, 0)
            pltpu.sync_copy(hist_ref.at[pl.ds(src0, C)], buf)
            pltpu.sync_copy(buf, out_ref.at[pl.ds(d0, C)])

    @pl.when(wid == 0)
    def _():
        pltpu.sync_copy(hval_ref, hv)
        pltpu.sync_copy(hv, out_ref.at[pl.ds(0, LAT)])

    @pl.when(jnp.logical_and(wid == SPLIT_E // P, ovf))
    def _():
        pltpu.sync_copy(hval_ref, hv)

        def initj(j, _):
            acc[pl.ds(j * 16, 16)] = hv[pl.ds(j * 16, 16)]
            return 0

        lax.fori_loop(0, LAT // 16, initj, 0)

        def body(r, _):
            pltpu.sync_copy(hist_ref.at[pl.ds(base + r * LAT, LAT)], row)

            def addj(j, _):
                acc[pl.ds(j * 16, 16)] = (
                    acc[pl.ds(j * 16, 16)] + row[pl.ds(j * 16, 16)]
                )
                return 0

            lax.fori_loop(0, LAT // 16, addj, 0)
            return 0

        lax.fori_loop(0, SPLIT - 1, body, 0)

        def finj(j, _):
            row[pl.ds(j * 16, 16)] = acc[pl.ds(j * 16, 16)] * (1.0 / SPLIT)
            return 0

        lax.fori_loop(0, LAT // 16, finj, 0)
        pltpu.sync_copy(row, out_ref.at[pl.ds(SPLIT_E, LAT)])


def _sc_call(histf, hvalf, scal32):
    mesh = plsc.VectorSubcoreMesh(
        core_axis_name="c", subcore_axis_name="s", num_cores=2, num_subcores=16
    )
    k = pl.kernel(
        _sc_body,
        out_type=jax.ShapeDtypeStruct((N,), jnp.float32),
        mesh=mesh,
        scratch_types=[
            pltpu.VMEM((C,), jnp.float32),
            pltpu.VMEM((32,), jnp.int32),
            pltpu.VMEM((LAT,), jnp.float32),
            pltpu.VMEM((LAT,), jnp.float32),
            pltpu.VMEM((LAT,), jnp.float32),
        ],
    )
    return k(histf, hvalf, scal32)


def kernel(hist, hist_time, hist_val, hist_time_val, counter, index):
    histf = hist.reshape(-1)
    hvalf = hist_val.reshape(-1)
    scal32 = jnp.concatenate(
        [
            jnp.asarray(index, jnp.int32).reshape(1),
            counter.reshape(-1),
            jnp.zeros((23,), jnp.int32),
        ]
    )
    out = _sc_call(histf, hvalf, scal32)
    return out.reshape(S, 1, LAT)
